# baseline (speedup 1.0000x reference)
import jax
import jax.numpy as jnp
from jax.experimental import pallas as pl

_N = 100000
_F = 128
_T = 32
_S = 32
_G = 128
_NB = 10000
_GRID = _N // _NB


def _body(x_ref, b_ref, v_ref, lin_ref, o_ref, c_ref):
    i = pl.program_id(0)
    nh25t = jax.lax.dot_general(
        v_ref[...], x_ref[...],
        (((0,), (1,)), ((), ())),
        preferred_element_type=jnp.float32) * 25.0
    nhb = nh25t.astype(jnp.bfloat16)
    gid = jax.lax.broadcasted_iota(jnp.int32, (_G, _NB), 0)
    oh = (gid == b_ref[0]).astype(jnp.bfloat16)
    cnt = jax.lax.dot_general(
        jnp.ones((8, _NB), jnp.bfloat16), oh, (((1,), (1,)), ((), ())),
        preferred_element_type=jnp.float32)
    rows = []
    for s in range(_S):
        z = (lin_ref[0, s] * 25.0).astype(jnp.bfloat16) - nhb
        rows.append(jnp.tanh(z))
    et = jnp.concatenate(rows, axis=0)
    acc = jax.lax.dot_general(
        et, oh, (((1,), (1,)), ((), ())),
        preferred_element_type=jnp.float32)

    @pl.when(i == 0)
    def _():
        o_ref[...] = jnp.zeros_like(o_ref)
        c_ref[...] = jnp.zeros_like(c_ref)

    o_ref[...] += acc
    c_ref[...] += cnt


@jax.jit
def kernel(x, batch, v, lin):
    b3 = batch.reshape(_GRID, 1, _NB)
    lin2 = lin.reshape(1, _S)
    x = x.astype(jnp.bfloat16)
    v = v.astype(jnp.bfloat16)
    acc, cnt = pl.pallas_call(
        _body,
        grid=(_GRID,),
        in_specs=[
            pl.BlockSpec((_NB, _F), lambda i: (i, 0)),
            pl.BlockSpec((1, 1, _NB), lambda i: (i, 0, 0)),
            pl.BlockSpec((_F, _T), lambda i: (0, 0)),
            pl.BlockSpec((1, _S), lambda i: (0, 0)),
        ],
        out_specs=[
            pl.BlockSpec((_S * _T, _G), lambda i: (0, 0)),
            pl.BlockSpec((8, _G), lambda i: (0, 0)),
        ],
        out_shape=[
            jax.ShapeDtypeStruct((_S * _T, _G), jnp.float32),
            jax.ShapeDtypeStruct((8, _G), jnp.float32),
        ],
    )(x, b3, v, lin2)
    out = 0.5 * acc.T + 0.5 * cnt[0][:, None]
    return out.reshape(_G, _S, _T)


# speedup vs baseline: 1.2960x; 1.2960x over previous
import jax
import jax.numpy as jnp
from jax.experimental import pallas as pl

_N = 100000
_F = 128
_T = 32
_S = 32
_G = 128
_NB = 10000
_GRID = _N // _NB


def _body(x_ref, b_ref, v_ref, lin_ref, o_ref, c_ref):
    i = pl.program_id(0)
    nh25t = jax.lax.dot_general(
        v_ref[...], x_ref[...],
        (((0,), (1,)), ((), ())),
        preferred_element_type=jnp.float32) * 25.0
    nhb = nh25t.astype(jnp.bfloat16)
    gid = jax.lax.broadcasted_iota(jnp.int32, (_G, _NB), 0)
    oh = (gid == b_ref[0]).astype(jnp.bfloat16)
    cnt = jax.lax.dot_general(
        jnp.ones((8, _NB), jnp.bfloat16), oh, (((1,), (1,)), ((), ())),
        preferred_element_type=jnp.float32)
    rows = []
    for s in range(_S):
        z = (lin_ref[0, s] * 25.0).astype(jnp.bfloat16) - nhb
        rows.append(jnp.tanh(z))
    et = jnp.concatenate(rows, axis=0)
    acc = jax.lax.dot_general(
        et, oh, (((1,), (1,)), ((), ())),
        preferred_element_type=jnp.float32)

    @pl.when(i == 0)
    def _():
        o_ref[...] = jnp.zeros_like(o_ref)
        c_ref[...] = jnp.zeros_like(c_ref)

    o_ref[...] += acc
    c_ref[...] += cnt


@jax.jit
def kernel(x, batch, v, lin):
    b3 = batch.reshape(_GRID, 1, _NB)
    lin2 = lin.reshape(1, _S)
    acc, cnt = pl.pallas_call(
        _body,
        grid=(_GRID,),
        in_specs=[
            pl.BlockSpec((_NB, _F), lambda i: (i, 0)),
            pl.BlockSpec((1, 1, _NB), lambda i: (i, 0, 0)),
            pl.BlockSpec((_F, _T), lambda i: (0, 0)),
            pl.BlockSpec((1, _S), lambda i: (0, 0)),
        ],
        out_specs=[
            pl.BlockSpec((_S * _T, _G), lambda i: (0, 0)),
            pl.BlockSpec((8, _G), lambda i: (0, 0)),
        ],
        out_shape=[
            jax.ShapeDtypeStruct((_S * _T, _G), jnp.float32),
            jax.ShapeDtypeStruct((8, _G), jnp.float32),
        ],
    )(x, b3, v, lin2)
    out = 0.5 * acc.T + 0.5 * cnt[0][:, None]
    return out.reshape(_G, _S, _T)


# speedup vs baseline: 1.2973x; 1.0010x over previous
import jax
import jax.numpy as jnp
from jax.experimental import pallas as pl

_N = 100000
_F = 128
_T = 32
_S = 32
_G = 128
_NB = 20000
_GRID = _N // _NB


def _body(x_ref, b_ref, v_ref, lin_ref, o_ref, c_ref):
    i = pl.program_id(0)
    nh25t = jax.lax.dot_general(
        v_ref[...], x_ref[...],
        (((0,), (1,)), ((), ())),
        preferred_element_type=jnp.float32) * 25.0
    nhb = nh25t.astype(jnp.bfloat16)
    gid = jax.lax.broadcasted_iota(jnp.int32, (_G, _NB), 0)
    oh = (gid == b_ref[0]).astype(jnp.bfloat16)
    cnt = jax.lax.dot_general(
        jnp.ones((8, _NB), jnp.bfloat16), oh, (((1,), (1,)), ((), ())),
        preferred_element_type=jnp.float32)
    rows = []
    for s in range(_S):
        z = (lin_ref[0, s] * 25.0).astype(jnp.bfloat16) - nhb
        rows.append(jnp.tanh(z))
    et = jnp.concatenate(rows, axis=0)
    acc = jax.lax.dot_general(
        et, oh, (((1,), (1,)), ((), ())),
        preferred_element_type=jnp.float32)

    @pl.when(i == 0)
    def _():
        o_ref[...] = jnp.zeros_like(o_ref)
        c_ref[...] = jnp.zeros_like(c_ref)

    o_ref[...] += acc
    c_ref[...] += cnt


@jax.jit
def kernel(x, batch, v, lin):
    b3 = batch.reshape(_GRID, 1, _NB)
    lin2 = lin.reshape(1, _S)
    acc, cnt = pl.pallas_call(
        _body,
        grid=(_GRID,),
        in_specs=[
            pl.BlockSpec((_NB, _F), lambda i: (i, 0)),
            pl.BlockSpec((1, 1, _NB), lambda i: (i, 0, 0)),
            pl.BlockSpec((_F, _T), lambda i: (0, 0)),
            pl.BlockSpec((1, _S), lambda i: (0, 0)),
        ],
        out_specs=[
            pl.BlockSpec((_S * _T, _G), lambda i: (0, 0)),
            pl.BlockSpec((8, _G), lambda i: (0, 0)),
        ],
        out_shape=[
            jax.ShapeDtypeStruct((_S * _T, _G), jnp.float32),
            jax.ShapeDtypeStruct((8, _G), jnp.float32),
        ],
    )(x, b3, v, lin2)
    out = 0.5 * acc.T + 0.5 * cnt[0][:, None]
    return out.reshape(_G, _S, _T)


# speedup vs baseline: 1.3010x; 1.0028x over previous
import jax
import jax.numpy as jnp
from jax.experimental import pallas as pl
from jax.experimental.pallas import tpu as pltpu

_N = 100000
_F = 128
_T = 32
_S = 32
_G = 128
_NB = 10000
_GRID = _N // _NB


def _body(x_ref, b_ref, v_ref, lin_ref, o_ref, acc_s, cnt_s):
    i = pl.program_id(0)
    nh25t = jax.lax.dot_general(
        v_ref[...], x_ref[...],
        (((0,), (1,)), ((), ())),
        preferred_element_type=jnp.float32) * 25.0
    nhb = nh25t.astype(jnp.bfloat16)
    gid = jax.lax.broadcasted_iota(jnp.int32, (_G, _NB), 0)
    oh = (gid == b_ref[0]).astype(jnp.bfloat16)
    cnt = jax.lax.dot_general(
        jnp.ones((8, _NB), jnp.bfloat16), oh, (((1,), (1,)), ((), ())),
        preferred_element_type=jnp.float32)
    rows = []
    for s in range(_S):
        z = (lin_ref[0, s] * 25.0).astype(jnp.bfloat16) - nhb
        rows.append(jnp.tanh(z))
    et = jnp.concatenate(rows, axis=0)
    acc = jax.lax.dot_general(
        et, oh, (((1,), (1,)), ((), ())),
        preferred_element_type=jnp.float32)

    @pl.when(i == 0)
    def _():
        acc_s[...] = jnp.zeros_like(acc_s)
        cnt_s[...] = jnp.zeros_like(cnt_s)

    acc_s[...] += acc
    cnt_s[...] += cnt

    @pl.when(i == _GRID - 1)
    def _():
        tot = acc_s[...].T
        o_ref[...] = 0.5 * tot + 0.5 * cnt_s[0, :][:, None]


@jax.jit
def kernel(x, batch, v, lin):
    b3 = batch.reshape(_GRID, 1, _NB)
    lin2 = lin.reshape(1, _S)
    out2 = pl.pallas_call(
        _body,
        grid=(_GRID,),
        in_specs=[
            pl.BlockSpec((_NB, _F), lambda i: (i, 0)),
            pl.BlockSpec((1, 1, _NB), lambda i: (i, 0, 0)),
            pl.BlockSpec((_F, _T), lambda i: (0, 0)),
            pl.BlockSpec((1, _S), lambda i: (0, 0)),
        ],
        out_specs=pl.BlockSpec((_G, _S * _T), lambda i: (0, 0)),
        out_shape=jax.ShapeDtypeStruct((_G, _S * _T), jnp.float32),
        scratch_shapes=[
            pltpu.VMEM((_S * _T, _G), jnp.float32),
            pltpu.VMEM((8, _G), jnp.float32),
        ],
    )(x, b3, v, lin2)
    return out2.reshape(_G, _S, _T)
